# baseline (speedup 1.0000x reference)
import functools

import jax
import jax.numpy as jnp
from jax import lax
from jax.experimental import pallas as pl
from jax.experimental.pallas import tpu as pltpu
from jax.experimental.pallas import tpu_sc as plsc


@functools.lru_cache(maxsize=None)
def _build_sc_kernel(B, NCLS, C, H, W):
    D = C * H * W
    info = plsc.get_sparse_core_info()
    NC, NS, L = info.num_cores, info.num_subcores, info.num_lanes
    NW = NC * NS
    BPW = B // NW
    U = 8
    CH = C * H

    mesh = plsc.VectorSubcoreMesh(core_axis_name="c", subcore_axis_name="s")

    @functools.partial(
        pl.kernel,
        mesh=mesh,
        out_type=jax.ShapeDtypeStruct((B, C, H, W), jnp.float32),
        scratch_types=[
            pltpu.VMEM((BPW * L,), jnp.int32),
            pltpu.VMEM((NCLS, D), jnp.float32),
            pltpu.VMEM((NCLS, D), jnp.float32),
            pltpu.VMEM((C, H, W), jnp.float32),
            pltpu.VMEM((C, H, W), jnp.float32),
            pltpu.SemaphoreType.DMA,
            pltpu.SemaphoreType.DMA,
            pltpu.SemaphoreType.DMA,
            pltpu.SemaphoreType.DMA,
        ],
    )
    def sc_fma(lab_hbm, mean_hbm, std_hbm, noise_hbm, out_hbm,
               idx_v, mtab, stab, nbuf0, nbuf1,
               in_sem0, in_sem1, out_sem0, out_sem1):
        wid = lax.axis_index("s") * NC + lax.axis_index("c")
        base = wid * BPW
        nbufs = (nbuf0, nbuf1)
        in_sems, out_sems = (in_sem0, in_sem1), (out_sem0, out_sem1)

        pltpu.sync_copy(lab_hbm.at[wid], idx_v)
        h_m = pltpu.async_copy(mean_hbm, mtab, out_sem0)
        h_s = pltpu.async_copy(std_hbm, stab, out_sem1)

        def issue_in(c, p):
            pltpu.async_copy(noise_hbm.at[base + c], nbufs[p], in_sems[p])

        def wait_in(p):
            pltpu.make_async_copy(noise_hbm.at[base], nbufs[p],
                                  in_sems[p]).wait()

        def issue_out(c, p):
            pltpu.async_copy(nbufs[p], out_hbm.at[base + c], out_sems[p])

        def wait_out(p):
            pltpu.make_async_copy(nbufs[p], out_hbm.at[base], out_sems[p]).wait()

        def compute(c, p):
            nb = nbufs[p]
            labv = idx_v[pl.ds(pl.multiple_of(c * L, L), L)]
            lab = labv[0]

            @plsc.parallel_loop(0, CH, step=1, unroll=U)
            def ch_body(i, lab=lab):
                ch = i // H
                h = i % H
                for wg in range(W // L):
                    sl = pl.ds(wg * L, L)
                    n = nb[ch, h, sl]
                    m = mtab[lab, pl.ds(i * W + wg * L, L)]
                    s = stab[lab, pl.ds(i * W + wg * L, L)]
                    nb[ch, h, sl] = m + s * n

        issue_in(0, 0)
        h_m.wait()
        h_s.wait()

        issue_in(1, 1)
        wait_in(0)
        compute(0, 0)
        issue_out(0, 0)

        def pair(i, _):
            for k in (1, 2):
                c = 2 * i + k
                p = k % 2
                wait_out(1 - p)
                issue_in(c + 1, 1 - p)
                wait_in(p)
                compute(c, p)
                issue_out(c, p)
            return 0

        lax.fori_loop(0, (BPW - 2) // 2, pair, 0)

        wait_out(0)
        wait_in(1)
        compute(BPW - 1, 1)
        issue_out(BPW - 1, 1)
        wait_out(1)

    return sc_fma, NW, BPW


def kernel(labels, class_means, class_stds, noise):
    B = labels.shape[0]
    NCLS, C, H, W = class_means.shape
    sc_fma, NW, BPW = _build_sc_kernel(B, NCLS, C, H, W)
    lab16 = jnp.repeat(labels.astype(jnp.int32), 16).reshape(NW, BPW * 16)
    return sc_fma(
        lab16,
        class_means.reshape(NCLS, C * H * W),
        class_stds.reshape(NCLS, C * H * W),
        noise,
    )


# speedup vs baseline: 1.0878x; 1.0878x over previous
import functools

import jax
import jax.numpy as jnp
from jax import lax
from jax.experimental import pallas as pl
from jax.experimental.pallas import tpu as pltpu
from jax.experimental.pallas import tpu_sc as plsc


@functools.lru_cache(maxsize=None)
def _build_sc_kernel(B, NCLS, C, H, W):
    D = C * H * W
    info = plsc.get_sparse_core_info()
    NC, NS, L = info.num_cores, info.num_subcores, info.num_lanes
    NW = NC * NS
    CH = C * H
    CHPW = CH // NW
    WT = 8
    BC = 1024
    NCHUNK = CHPW * (W // WT) * (B // BC)
    U = 2

    mesh = plsc.VectorSubcoreMesh(core_axis_name="c", subcore_axis_name="s")

    @functools.partial(
        pl.kernel,
        mesh=mesh,
        out_type=jax.ShapeDtypeStruct((C, H, W, B), jnp.float32),
        compiler_params=pltpu.CompilerParams(needs_layout_passes=False),
        scratch_types=[
            pltpu.VMEM((B,), jnp.int32),
            pltpu.VMEM((NCLS * D,), jnp.float32),
            pltpu.VMEM((NCLS * D,), jnp.float32),
            pltpu.VMEM((WT, BC), jnp.float32),
            pltpu.VMEM((WT, BC), jnp.float32),
            pltpu.SemaphoreType.DMA,
            pltpu.SemaphoreType.DMA,
            pltpu.SemaphoreType.DMA,
            pltpu.SemaphoreType.DMA,
        ],
    )
    def sc_fma(lab_hbm, mean_hbm, std_hbm, noise_hbm, out_hbm,
               idx_v, mtab, stab, nbuf0, nbuf1,
               in_sem0, in_sem1, out_sem0, out_sem1):
        nv = noise_hbm.reshape(CH, W, B)
        ov = out_hbm.reshape(CH, W, B)
        wid = lax.axis_index("s") * NC + lax.axis_index("c")
        base_ch = wid * CHPW
        nbufs = (nbuf0, nbuf1)
        in_sems, out_sems = (in_sem0, in_sem1), (out_sem0, out_sem1)

        pltpu.sync_copy(lab_hbm, idx_v)
        h_m = pltpu.async_copy(mean_hbm, mtab, out_sem0)
        h_s = pltpu.async_copy(std_hbm, stab, out_sem1)

        def coords(c):
            chrow = base_ch + c // 16
            w0 = ((c // 4) % 4) * WT
            b0 = (c % 4) * BC
            return chrow, w0, b0

        def issue_in(c, p):
            chrow, w0, b0 = coords(c)
            pltpu.async_copy(nv.at[chrow, pl.ds(w0, WT), pl.ds(b0, BC)],
                             nbufs[p], in_sems[p])

        def wait_in(p):
            pltpu.make_async_copy(nv.at[0, pl.ds(0, WT), pl.ds(0, BC)],
                                  nbufs[p], in_sems[p]).wait()

        def issue_out(c, p):
            chrow, w0, b0 = coords(c)
            pltpu.async_copy(nbufs[p], ov.at[chrow, pl.ds(w0, WT), pl.ds(b0, BC)],
                             out_sems[p])

        def wait_out(p):
            pltpu.make_async_copy(nbufs[p], ov.at[0, pl.ds(0, WT), pl.ds(0, BC)],
                                  out_sems[p]).wait()

        def compute(c, p):
            nb = nbufs[p]
            chrow, w0, b0 = coords(c)
            dbase = chrow * W + w0
            dvecs = [jnp.full((L,), 1, jnp.int32) * (dbase + wi)
                     for wi in range(WT)]

            @plsc.parallel_loop(0, BC // L, step=1, unroll=U)
            def win_body(t):
                labv = idx_v[pl.ds(b0 + t * L, L)]
                sl = pl.ds(t * L, L)
                for wi in range(WT):
                    addr = labv + dvecs[wi]
                    n = nb[wi, sl]
                    m = plsc.load_gather(mtab, [addr])
                    s = plsc.load_gather(stab, [addr])
                    nb[wi, sl] = m + s * n

        issue_in(0, 0)
        h_m.wait()
        h_s.wait()

        issue_in(1, 1)
        wait_in(0)
        compute(0, 0)
        issue_out(0, 0)

        def pair(i, _):
            for k in (1, 2):
                c = 2 * i + k
                p = k % 2
                wait_out(1 - p)
                issue_in(c + 1, 1 - p)
                wait_in(p)
                compute(c, p)
                issue_out(c, p)
            return 0

        lax.fori_loop(0, (NCHUNK - 2) // 2, pair, 0)

        wait_out(0)
        wait_in(1)
        compute(NCHUNK - 1, 1)
        issue_out(NCHUNK - 1, 1)
        wait_out(1)

    return sc_fma


def kernel(labels, class_means, class_stds, noise):
    B = labels.shape[0]
    NCLS, C, H, W = class_means.shape
    D = C * H * W
    sc_fma = _build_sc_kernel(B, NCLS, C, H, W)
    out_t = sc_fma(
        labels.astype(jnp.int32) * D,
        class_means.reshape(NCLS * D),
        class_stds.reshape(NCLS * D),
        jnp.transpose(noise, (1, 2, 3, 0)),
    )
    return jnp.transpose(out_t, (3, 0, 1, 2))


# speedup vs baseline: 4.9681x; 4.5670x over previous
import functools

import jax
import jax.numpy as jnp
from jax import lax
from jax.experimental import pallas as pl
from jax.experimental.pallas import tpu as pltpu
from jax.experimental.pallas import tpu_sc as plsc


@functools.lru_cache(maxsize=None)
def _build_sc_kernel(B, NCLS, C, H, W):
    D = C * H * W
    info = plsc.get_sparse_core_info()
    NC, NS, L = info.num_cores, info.num_subcores, info.num_lanes
    NW = NC * NS
    CH = C * H
    CHPW = CH // NW
    WT = 8
    BC = 1024
    NCHUNK = CHPW * (W // WT) * (B // BC)
    U = 2

    mesh = plsc.VectorSubcoreMesh(core_axis_name="c", subcore_axis_name="s")

    @functools.partial(
        pl.kernel,
        mesh=mesh,
        out_type=jax.ShapeDtypeStruct((C, H, W, B), jnp.float32),
        compiler_params=pltpu.CompilerParams(needs_layout_passes=False),
        scratch_types=[
            pltpu.VMEM((B,), jnp.int32),
            pltpu.VMEM((NCLS * (D + 1),), jnp.float32),
            pltpu.VMEM((NCLS * (D + 1),), jnp.float32),
            pltpu.VMEM((WT, BC), jnp.float32),
            pltpu.VMEM((WT, BC), jnp.float32),
            pltpu.SemaphoreType.DMA,
            pltpu.SemaphoreType.DMA,
            pltpu.SemaphoreType.DMA,
            pltpu.SemaphoreType.DMA,
        ],
    )
    def sc_fma(lab_hbm, mean_hbm, std_hbm, noise_hbm, out_hbm,
               idx_v, mtab, stab, nbuf0, nbuf1,
               in_sem0, in_sem1, out_sem0, out_sem1):
        nv = noise_hbm.reshape(CH, W, B)
        ov = out_hbm.reshape(CH, W, B)
        wid = lax.axis_index("s") * NC + lax.axis_index("c")
        base_ch = wid * CHPW
        nbufs = (nbuf0, nbuf1)
        in_sems, out_sems = (in_sem0, in_sem1), (out_sem0, out_sem1)

        pltpu.sync_copy(lab_hbm, idx_v)
        h_m = pltpu.async_copy(mean_hbm, mtab, out_sem0)
        h_s = pltpu.async_copy(std_hbm, stab, out_sem1)

        def coords(c):
            chrow = base_ch + c // 16
            w0 = ((c // 4) % 4) * WT
            b0 = (c % 4) * BC
            return chrow, w0, b0

        def issue_in(c, p):
            chrow, w0, b0 = coords(c)
            pltpu.async_copy(nv.at[chrow, pl.ds(w0, WT), pl.ds(b0, BC)],
                             nbufs[p], in_sems[p])

        def wait_in(p):
            pltpu.make_async_copy(nv.at[0, pl.ds(0, WT), pl.ds(0, BC)],
                                  nbufs[p], in_sems[p]).wait()

        def issue_out(c, p):
            chrow, w0, b0 = coords(c)
            pltpu.async_copy(nbufs[p], ov.at[chrow, pl.ds(w0, WT), pl.ds(b0, BC)],
                             out_sems[p])

        def wait_out(p):
            pltpu.make_async_copy(nbufs[p], ov.at[0, pl.ds(0, WT), pl.ds(0, BC)],
                                  out_sems[p]).wait()

        def compute(c, p):
            nb = nbufs[p]
            chrow, w0, b0 = coords(c)
            dbase = chrow * W + w0
            dvecs = [jnp.full((L,), 1, jnp.int32) * (dbase + wi)
                     for wi in range(WT)]

            @plsc.parallel_loop(0, BC // L, step=1, unroll=U)
            def win_body(t):
                labv = idx_v[pl.ds(b0 + t * L, L)]
                sl = pl.ds(t * L, L)
                for wi in range(WT):
                    addr = labv + dvecs[wi]
                    n = nb[wi, sl]
                    m = plsc.load_gather(mtab, [addr])
                    s = plsc.load_gather(stab, [addr])
                    nb[wi, sl] = m + s * n

        issue_in(0, 0)
        h_m.wait()
        h_s.wait()

        issue_in(1, 1)
        wait_in(0)
        compute(0, 0)
        issue_out(0, 0)

        def pair(i, _):
            for k in (1, 2):
                c = 2 * i + k
                p = k % 2
                wait_out(1 - p)
                issue_in(c + 1, 1 - p)
                wait_in(p)
                compute(c, p)
                issue_out(c, p)
            return 0

        lax.fori_loop(0, (NCHUNK - 2) // 2, pair, 0)

        wait_out(0)
        wait_in(1)
        compute(NCHUNK - 1, 1)
        issue_out(NCHUNK - 1, 1)
        wait_out(1)

    return sc_fma


def kernel(labels, class_means, class_stds, noise):
    B = labels.shape[0]
    NCLS, C, H, W = class_means.shape
    D = C * H * W
    sc_fma = _build_sc_kernel(B, NCLS, C, H, W)
    pad_mean = jnp.pad(class_means.reshape(NCLS, D), ((0, 0), (0, 1)))
    pad_std = jnp.pad(class_stds.reshape(NCLS, D), ((0, 0), (0, 1)))
    out_t = sc_fma(
        labels.astype(jnp.int32) * (D + 1),
        pad_mean.reshape(NCLS * (D + 1)),
        pad_std.reshape(NCLS * (D + 1)),
        jnp.transpose(noise, (1, 2, 3, 0)),
    )
    return jnp.transpose(out_t, (3, 0, 1, 2))


# speedup vs baseline: 5.0882x; 1.0242x over previous
import functools

import jax
import jax.numpy as jnp
from jax import lax
from jax.experimental import pallas as pl
from jax.experimental.pallas import tpu as pltpu
from jax.experimental.pallas import tpu_sc as plsc


@functools.lru_cache(maxsize=None)
def _build_sc_kernel(B, NCLS, C, H, W):
    D = C * H * W
    info = plsc.get_sparse_core_info()
    NC, NS, L = info.num_cores, info.num_subcores, info.num_lanes
    NW = NC * NS
    CH = C * H
    CHPW = CH // NW
    WT = 8
    BC = 2048
    NCHUNK = CHPW * (W // WT) * (B // BC)
    U = 4

    mesh = plsc.VectorSubcoreMesh(core_axis_name="c", subcore_axis_name="s")

    @functools.partial(
        pl.kernel,
        mesh=mesh,
        out_type=jax.ShapeDtypeStruct((C, H, W, B), jnp.float32),
        compiler_params=pltpu.CompilerParams(needs_layout_passes=False),
        scratch_types=[
            pltpu.VMEM((B,), jnp.int32),
            pltpu.VMEM((NCLS * (D + 1),), jnp.float32),
            pltpu.VMEM((NCLS * (D + 1),), jnp.float32),
            pltpu.VMEM((WT, BC), jnp.float32),
            pltpu.VMEM((WT, BC), jnp.float32),
            pltpu.SemaphoreType.DMA,
            pltpu.SemaphoreType.DMA,
            pltpu.SemaphoreType.DMA,
            pltpu.SemaphoreType.DMA,
        ],
    )
    def sc_fma(lab_hbm, mean_hbm, std_hbm, noise_hbm, out_hbm,
               idx_v, mtab, stab, nbuf0, nbuf1,
               in_sem0, in_sem1, out_sem0, out_sem1):
        nv = noise_hbm.reshape(CH, W, B)
        ov = out_hbm.reshape(CH, W, B)
        wid = lax.axis_index("s") * NC + lax.axis_index("c")
        base_ch = wid * CHPW
        nbufs = (nbuf0, nbuf1)
        in_sems, out_sems = (in_sem0, in_sem1), (out_sem0, out_sem1)

        pltpu.sync_copy(lab_hbm, idx_v)
        h_m = pltpu.async_copy(mean_hbm, mtab, out_sem0)
        h_s = pltpu.async_copy(std_hbm, stab, out_sem1)

        NB = B // BC
        NWT = W // WT

        def coords(c):
            chrow = base_ch + c // (NB * NWT)
            w0 = ((c // NB) % NWT) * WT
            b0 = (c % NB) * BC
            return chrow, w0, b0

        def issue_in(c, p):
            chrow, w0, b0 = coords(c)
            pltpu.async_copy(nv.at[chrow, pl.ds(w0, WT), pl.ds(b0, BC)],
                             nbufs[p], in_sems[p])

        def wait_in(p):
            pltpu.make_async_copy(nv.at[0, pl.ds(0, WT), pl.ds(0, BC)],
                                  nbufs[p], in_sems[p]).wait()

        def issue_out(c, p):
            chrow, w0, b0 = coords(c)
            pltpu.async_copy(nbufs[p], ov.at[chrow, pl.ds(w0, WT), pl.ds(b0, BC)],
                             out_sems[p])

        def wait_out(p):
            pltpu.make_async_copy(nbufs[p], ov.at[0, pl.ds(0, WT), pl.ds(0, BC)],
                                  out_sems[p]).wait()

        def compute(c, p):
            nb = nbufs[p]
            chrow, w0, b0 = coords(c)
            dbase = chrow * W + w0
            dvecs = [jnp.full((L,), 1, jnp.int32) * (dbase + wi)
                     for wi in range(WT)]

            @plsc.parallel_loop(0, BC // L, step=1, unroll=U)
            def win_body(t):
                labv = idx_v[pl.ds(b0 + t * L, L)]
                sl = pl.ds(t * L, L)
                for wi in range(WT):
                    addr = labv + dvecs[wi]
                    n = nb[wi, sl]
                    m = plsc.load_gather(mtab, [addr])
                    s = plsc.load_gather(stab, [addr])
                    nb[wi, sl] = m + s * n

        issue_in(0, 0)
        h_m.wait()
        h_s.wait()

        issue_in(1, 1)
        wait_in(0)
        compute(0, 0)
        issue_out(0, 0)

        def pair(i, _):
            for k in (1, 2):
                c = 2 * i + k
                p = k % 2
                wait_out(1 - p)
                issue_in(c + 1, 1 - p)
                wait_in(p)
                compute(c, p)
                issue_out(c, p)
            return 0

        lax.fori_loop(0, (NCHUNK - 2) // 2, pair, 0)

        wait_out(0)
        wait_in(1)
        compute(NCHUNK - 1, 1)
        issue_out(NCHUNK - 1, 1)
        wait_out(1)

    return sc_fma


def kernel(labels, class_means, class_stds, noise):
    B = labels.shape[0]
    NCLS, C, H, W = class_means.shape
    D = C * H * W
    sc_fma = _build_sc_kernel(B, NCLS, C, H, W)
    pad_mean = jnp.pad(class_means.reshape(NCLS, D), ((0, 0), (0, 1)))
    pad_std = jnp.pad(class_stds.reshape(NCLS, D), ((0, 0), (0, 1)))
    out_t = sc_fma(
        labels.astype(jnp.int32) * (D + 1),
        pad_mean.reshape(NCLS * (D + 1)),
        pad_std.reshape(NCLS * (D + 1)),
        jnp.transpose(noise, (1, 2, 3, 0)),
    )
    return jnp.transpose(out_t, (3, 0, 1, 2))


# speedup vs baseline: 5.8906x; 1.1577x over previous
import functools

import jax
import jax.numpy as jnp
from jax import lax
from jax.experimental import pallas as pl
from jax.experimental.pallas import tpu as pltpu
from jax.experimental.pallas import tpu_sc as plsc


@functools.lru_cache(maxsize=None)
def _build_sc_kernel(B, NCLS, C, H, W):
    D = C * H * W
    info = plsc.get_sparse_core_info()
    NC, NS, L = info.num_cores, info.num_subcores, info.num_lanes
    NW = NC * NS
    CH = C * H
    CHPW = CH // NW
    WT = 8
    BC = 1024
    NBUF = 4
    NB = B // BC
    NWT = W // WT
    NCHUNK = CHPW * NWT * NB
    U = 4

    mesh = plsc.VectorSubcoreMesh(core_axis_name="c", subcore_axis_name="s")

    @functools.partial(
        pl.kernel,
        mesh=mesh,
        out_type=jax.ShapeDtypeStruct((C, H, W, B), jnp.float32),
        compiler_params=pltpu.CompilerParams(needs_layout_passes=False),
        scratch_types=[
            pltpu.VMEM((B,), jnp.int32),
            pltpu.VMEM((NCLS * (D + 1),), jnp.float32),
            pltpu.VMEM((NCLS * (D + 1),), jnp.float32),
        ] + [pltpu.VMEM((WT, BC), jnp.float32) for _ in range(NBUF)]
          + [pltpu.SemaphoreType.DMA for _ in range(2 * NBUF)],
    )
    def sc_fma(lab_hbm, mean_hbm, std_hbm, noise_hbm, out_hbm,
               idx_v, mtab, stab, nbuf0, nbuf1, nbuf2, nbuf3,
               in_sem0, in_sem1, in_sem2, in_sem3,
               out_sem0, out_sem1, out_sem2, out_sem3):
        nv = noise_hbm.reshape(CH, W, B)
        ov = out_hbm.reshape(CH, W, B)
        wid = lax.axis_index("s") * NC + lax.axis_index("c")
        base_ch = wid * CHPW
        nbufs = (nbuf0, nbuf1, nbuf2, nbuf3)
        in_sems = (in_sem0, in_sem1, in_sem2, in_sem3)
        out_sems = (out_sem0, out_sem1, out_sem2, out_sem3)

        pltpu.sync_copy(lab_hbm, idx_v)
        h_m = pltpu.async_copy(mean_hbm, mtab, out_sem0)
        h_s = pltpu.async_copy(std_hbm, stab, out_sem1)

        def coords(c):
            chrow = base_ch + c // (NB * NWT)
            w0 = ((c // NB) % NWT) * WT
            b0 = (c % NB) * BC
            return chrow, w0, b0

        def issue_in(c, p):
            chrow, w0, b0 = coords(c)
            pltpu.async_copy(nv.at[chrow, pl.ds(w0, WT), pl.ds(b0, BC)],
                             nbufs[p], in_sems[p])

        def wait_in(p):
            pltpu.make_async_copy(nv.at[0, pl.ds(0, WT), pl.ds(0, BC)],
                                  nbufs[p], in_sems[p]).wait()

        def issue_out(c, p):
            chrow, w0, b0 = coords(c)
            pltpu.async_copy(nbufs[p], ov.at[chrow, pl.ds(w0, WT), pl.ds(b0, BC)],
                             out_sems[p])

        def wait_out(p):
            pltpu.make_async_copy(nbufs[p], ov.at[0, pl.ds(0, WT), pl.ds(0, BC)],
                                  out_sems[p]).wait()

        def compute(c, p):
            nb = nbufs[p]
            chrow, w0, b0 = coords(c)
            dbase = chrow * W + w0
            dvecs = [jnp.full((L,), 1, jnp.int32) * (dbase + wi)
                     for wi in range(WT)]

            @plsc.parallel_loop(0, BC // L, step=1, unroll=U)
            def win_body(t):
                labv = idx_v[pl.ds(b0 + t * L, L)]
                sl = pl.ds(t * L, L)
                for wi in range(WT):
                    addr = labv + dvecs[wi]
                    n = nb[wi, sl]
                    m = plsc.load_gather(mtab, [addr])
                    s = plsc.load_gather(stab, [addr])
                    nb[wi, sl] = m + s * n

        issue_in(0, 0)
        issue_in(1, 1)
        h_m.wait()
        h_s.wait()

        for c in (0, 1):
            issue_in(c + 2, c + 2)
            wait_in(c)
            compute(c, c)
            issue_out(c, c)

        def quad(i, _):
            for j in range(4):
                c = 4 * i + 2 + j
                p = (2 + j) % NBUF
                q = j % NBUF
                wait_out(q)
                issue_in(c + 2, q)
                wait_in(p)
                compute(c, p)
                issue_out(c, p)
            return 0

        lax.fori_loop(0, (NCHUNK - 4) // 4, quad, 0)

        for c in (NCHUNK - 2, NCHUNK - 1):
            p = c % NBUF
            wait_in(p)
            compute(c, p)
            issue_out(c, p)

        for p in range(NBUF):
            wait_out(p)

    return sc_fma


def kernel(labels, class_means, class_stds, noise):
    B = labels.shape[0]
    NCLS, C, H, W = class_means.shape
    D = C * H * W
    sc_fma = _build_sc_kernel(B, NCLS, C, H, W)
    pad_mean = jnp.pad(class_means.reshape(NCLS, D), ((0, 0), (0, 1)))
    pad_std = jnp.pad(class_stds.reshape(NCLS, D), ((0, 0), (0, 1)))
    out_t = sc_fma(
        labels.astype(jnp.int32) * (D + 1),
        pad_mean.reshape(NCLS * (D + 1)),
        pad_std.reshape(NCLS * (D + 1)),
        jnp.transpose(noise, (1, 2, 3, 0)),
    )
    return jnp.transpose(out_t, (3, 0, 1, 2))
